# baseline (speedup 1.0000x reference)
import functools

import jax
import jax.numpy as jnp
from jax import lax
from jax.experimental import pallas as pl
from jax.experimental.pallas import tpu as pltpu
from jax.experimental.pallas import tpu_sc as plsc

EMBED = 64
BATCH = 16384
OUT_W = 5 * EMBED

_info = plsc.get_sparse_core_info()
_NW = _info.num_cores * _info.num_subcores
_BPW = BATCH // _NW
_CH = 128
_NCH = _BPW // _CH

_mesh = plsc.VectorSubcoreMesh(core_axis_name="c", subcore_axis_name="s")
_params = pltpu.CompilerParams(use_tc_tiling_on_sc=True,
                               needs_layout_passes=False)


def _lookup_body(idx_hbm, tables, out, idx_v, rows_v, stage_v, sem, row0):
    nf = len(tables)
    wid = lax.axis_index("s") * _info.num_cores + lax.axis_index("c")
    base = wid * _BPW
    for t in range(nf):
        pltpu.sync_copy(idx_hbm[t].at[pl.ds(base, _BPW)], idx_v[t])

    lanes = jax.lax.iota(jnp.int32, 16)
    _dsplat = [jnp.full((16,), d, jnp.int32) for d in range(EMBED)]

    def issue(g, _, t, c, buf):
        v = idx_v[t][pl.ds(c * _CH + g * 16, 16)]
        for lane in range(16):
            pltpu.async_copy(
                tables[t].at[pl.ds(v[lane], 1), :],
                rows_v[buf].at[pl.ds(g * 16 + lane, 1), :],
                sem[buf])
        return 0

    def drain(g, _, buf):
        for lane in range(16):
            pltpu.make_async_copy(
                tables[0].at[pl.ds(0, 1), :],
                rows_v[0].at[pl.ds(0, 1), :],
                sem[buf]).wait()
        return 0

    def chunk(c, _):
        def field(t, buf):
            lax.fori_loop(0, _CH // 16,
                          functools.partial(issue, t=t, c=c, buf=buf), 0)

        field(0, 0)

        for t in range(nf):
            lax.fori_loop(0, _CH // 16,
                          functools.partial(drain, buf=t % 2), 0)
            if t + 1 < nf:
                field(t + 1, (t + 1) % 2)

            def asm(ug, _, t=t, buf=t % 2):
                users = ug * 16 + lanes
                for d in range(EMBED):
                    v = plsc.load_gather(rows_v[buf], [users, _dsplat[d]])
                    stage_v[t * EMBED + d, pl.ds(ug * 16, 16)] = v
                return 0
            lax.fori_loop(0, _CH // 16, asm, 0)

        pltpu.sync_copy(
            stage_v,
            out.at[pl.ds(row0, nf * EMBED), pl.ds(base + c * _CH, _CH)])
        return 0

    lax.fori_loop(0, _NCH, chunk, 0)


@functools.partial(
    pl.kernel,
    mesh=_mesh,
    out_type=(),
    scratch_types=[
        [pltpu.VMEM((_BPW,), jnp.int32) for _ in range(4)],
        [pltpu.VMEM((_CH, EMBED), jnp.float32) for _ in range(2)],
        pltpu.VMEM((4 * EMBED, _CH), jnp.float32),
        [pltpu.SemaphoreType.DMA for _ in range(2)],
    ],
    compiler_params=_params,
    name="small_fields",
)
def _small_fields(ep, pop, yr, st, et, pt, yt, stt, out,
                  idx_v, rows_v, stage_v, sem):
    _lookup_body([ep, pop, yr, st], [et, pt, yt, stt], out,
                 idx_v, rows_v, stage_v, sem, EMBED)


@functools.partial(
    pl.kernel,
    mesh=_mesh,
    out_type=(),
    scratch_types=[
        [pltpu.VMEM((_BPW,), jnp.int32)],
        [pltpu.VMEM((_CH, EMBED), jnp.float32) for _ in range(2)],
        pltpu.VMEM((EMBED, _CH), jnp.float32),
        [pltpu.SemaphoreType.DMA for _ in range(2)],
    ],
    compiler_params=_params,
    name="user_field",
)
def _user_field(uid, ut, out, idx_v, rows_v, stage_v, sem):
    _lookup_body([uid], [ut], out, idx_v, rows_v, stage_v, sem, 0)


def kernel(user_id, episodes, popularity, year, studio,
           user_table, episodes_table, popularity_table, year_table, studio_table):
    o_ref = jax.new_ref(pl.empty((OUT_W, BATCH), jnp.float32))
    _small_fields(episodes, popularity, year, studio,
                  episodes_table, popularity_table, year_table, studio_table,
                  o_ref)
    _user_field(user_id, user_table, o_ref)
    return o_ref[...].T


# speedup vs baseline: 1.1553x; 1.1553x over previous
import functools

import jax
import jax.numpy as jnp
from jax import lax
from jax.experimental import pallas as pl
from jax.experimental.pallas import tpu as pltpu
from jax.experimental.pallas import tpu_sc as plsc

EMBED = 64
BATCH = 16384
OUT_W = 5 * EMBED

_info = plsc.get_sparse_core_info()
_NW = _info.num_cores * _info.num_subcores
_BPW = BATCH // _NW
_CH = 128
_NCH = _BPW // _CH

_mesh = plsc.VectorSubcoreMesh(core_axis_name="c", subcore_axis_name="s")
_params = pltpu.CompilerParams(use_tc_tiling_on_sc=True,
                               needs_layout_passes=False)


def _lookup_body(idx_hbm, tables, out, idx_v, rows_v, stage_v, sem, row0):
    nf = len(tables)
    wid = lax.axis_index("s") * _info.num_cores + lax.axis_index("c")
    base = wid * _BPW
    for t in range(nf):
        pltpu.sync_copy(idx_hbm[t].at[pl.ds(base, _BPW)], idx_v[t])

    lanes = jax.lax.iota(jnp.int32, 16)
    _rowvec = [[t * EMBED + d0 + lanes for d0 in range(0, EMBED, 16)]
               for t in range(nf)]

    def issue(g, _, t, c, buf):
        v = idx_v[t][pl.ds(c * _CH + g * 16, 16)]
        for lane in range(16):
            pltpu.async_copy(
                tables[t].at[pl.ds(v[lane], 1), :],
                rows_v[buf].at[pl.ds(g * 16 + lane, 1), :],
                sem[buf])
        return 0

    def drain(g, _, buf):
        for lane in range(16):
            pltpu.make_async_copy(
                tables[0].at[pl.ds(0, 1), :],
                rows_v[0].at[pl.ds(0, 1), :],
                sem[buf]).wait()
        return 0

    def chunk(c, _):
        def field(t, buf):
            lax.fori_loop(0, _CH // 16,
                          functools.partial(issue, t=t, c=c, buf=buf), 0)

        field(0, 0)

        for t in range(nf):
            lax.fori_loop(0, _CH // 16,
                          functools.partial(drain, buf=t % 2), 0)
            if t + 1 < nf:
                field(t + 1, (t + 1) % 2)

            def asm(u, _, t=t, buf=t % 2):
                ucol = jnp.full((16,), u, jnp.int32)
                for g in range(EMBED // 16):
                    v = rows_v[buf][u, pl.ds(g * 16, 16)]
                    plsc.store_scatter(stage_v, [_rowvec[t][g], ucol], v)
                return 0
            lax.fori_loop(0, _CH, asm, 0)

        pltpu.sync_copy(
            stage_v,
            out.at[pl.ds(row0, nf * EMBED), pl.ds(base + c * _CH, _CH)])
        return 0

    lax.fori_loop(0, _NCH, chunk, 0)


@functools.partial(
    pl.kernel,
    mesh=_mesh,
    out_type=(),
    scratch_types=[
        [pltpu.VMEM((_BPW,), jnp.int32) for _ in range(4)],
        [pltpu.VMEM((_CH, EMBED), jnp.float32) for _ in range(2)],
        pltpu.VMEM((4 * EMBED, _CH), jnp.float32),
        [pltpu.SemaphoreType.DMA for _ in range(2)],
    ],
    compiler_params=_params,
    name="small_fields",
)
def _small_fields(ep, pop, yr, st, et, pt, yt, stt, out,
                  idx_v, rows_v, stage_v, sem):
    _lookup_body([ep, pop, yr, st], [et, pt, yt, stt], out,
                 idx_v, rows_v, stage_v, sem, EMBED)


@functools.partial(
    pl.kernel,
    mesh=_mesh,
    out_type=(),
    scratch_types=[
        [pltpu.VMEM((_BPW,), jnp.int32)],
        [pltpu.VMEM((_CH, EMBED), jnp.float32) for _ in range(2)],
        pltpu.VMEM((EMBED, _CH), jnp.float32),
        [pltpu.SemaphoreType.DMA for _ in range(2)],
    ],
    compiler_params=_params,
    name="user_field",
)
def _user_field(uid, ut, out, idx_v, rows_v, stage_v, sem):
    _lookup_body([uid], [ut], out, idx_v, rows_v, stage_v, sem, 0)


def kernel(user_id, episodes, popularity, year, studio,
           user_table, episodes_table, popularity_table, year_table, studio_table):
    o_ref = jax.new_ref(pl.empty((OUT_W, BATCH), jnp.float32))
    _small_fields(episodes, popularity, year, studio,
                  episodes_table, popularity_table, year_table, studio_table,
                  o_ref)
    _user_field(user_id, user_table, o_ref)
    return o_ref[...].T


# speedup vs baseline: 1.3023x; 1.1272x over previous
import functools

import jax
import jax.numpy as jnp
from jax import lax
from jax.experimental import pallas as pl
from jax.experimental.pallas import tpu as pltpu
from jax.experimental.pallas import tpu_sc as plsc

EMBED = 64
BATCH = 16384

_info = plsc.get_sparse_core_info()
_NW = _info.num_cores * _info.num_subcores
_BPW = BATCH // _NW
_CH = 128

_mesh = plsc.VectorSubcoreMesh(core_axis_name="c", subcore_axis_name="s")


@functools.partial(
    pl.kernel,
    mesh=_mesh,
    out_type=jax.ShapeDtypeStruct((BATCH, EMBED), jnp.float32),
    scratch_types=[
        pltpu.VMEM((_BPW,), jnp.int32),
        [pltpu.VMEM((_CH, EMBED), jnp.float32) for _ in range(2)],
        [pltpu.SemaphoreType.DMA for _ in range(2)],
    ],
    compiler_params=pltpu.CompilerParams(use_tc_tiling_on_sc=True,
                                         needs_layout_passes=False),
    name="user_field",
)
def _user_field(uid, ut, out, idx_v, rows_v, sem):
    wid = lax.axis_index("s") * _info.num_cores + lax.axis_index("c")
    base = wid * _BPW
    pltpu.sync_copy(uid.at[pl.ds(base, _BPW)], idx_v)

    def issue(g, _, c, buf):
        v = idx_v[pl.ds(c * _CH + g * 16, 16)]
        for lane in range(16):
            pltpu.async_copy(
                ut.at[pl.ds(v[lane], 1), :],
                rows_v[buf].at[pl.ds(g * 16 + lane, 1), :],
                sem[buf])
        return 0

    def drain(g, _, buf):
        for lane in range(16):
            pltpu.make_async_copy(
                ut.at[pl.ds(0, 1), :],
                rows_v[0].at[pl.ds(0, 1), :],
                sem[buf]).wait()
        return 0

    def chunk(c):
        lax.fori_loop(0, _CH // 16, functools.partial(issue, c=c, buf=c % 2), 0)

    n_chunks = _BPW // _CH
    chunk(0)
    for c in range(n_chunks):
        if c + 1 < n_chunks:
            chunk(c + 1)
        lax.fori_loop(0, _CH // 16, functools.partial(drain, buf=c % 2), 0)
        pltpu.sync_copy(rows_v[c % 2], out.at[pl.ds(base + c * _CH, _CH), :])


@functools.partial(
    pl.kernel,
    mesh=_mesh,
    out_type=jax.ShapeDtypeStruct((BATCH, 4 * EMBED), jnp.float32),
    scratch_types=[
        [pltpu.VMEM((_BPW,), jnp.int32) for _ in range(4)],
        [pltpu.VMEM((_BPW, EMBED), jnp.float32) for _ in range(2)],
        pltpu.SemaphoreType.DMA,
    ],
    compiler_params=pltpu.CompilerParams(use_tc_tiling_on_sc=False),
    name="small_fields",
)
def _small_fields(ep, pop, yr, st, et, pt, yt, stt, out, idx_v, rows_v, gsem):
    wid = lax.axis_index("s") * _info.num_cores + lax.axis_index("c")
    base = wid * _BPW
    idx_hbm = [ep, pop, yr, st]
    tables = [et, pt, yt, stt]
    for t in range(4):
        pltpu.sync_copy(idx_hbm[t].at[pl.ds(base, _BPW)], idx_v[t])

    def start_gather(t, buf):
        return pltpu.async_copy(tables[t].at[idx_v[t]], rows_v[buf], gsem)

    cp = start_gather(0, 0)
    for t in range(4):
        cp.wait()
        if t + 1 < 4:
            nxt = start_gather(t + 1, (t + 1) % 2)
        pltpu.sync_copy(
            rows_v[t % 2],
            out.at[pl.ds(base, _BPW), pl.ds(t * EMBED, EMBED)])
        if t + 1 < 4:
            cp = nxt


def kernel(user_id, episodes, popularity, year, studio,
           user_table, episodes_table, popularity_table, year_table, studio_table):
    user_part = _user_field(user_id, user_table)
    small_part = _small_fields(episodes, popularity, year, studio,
                               episodes_table, popularity_table, year_table,
                               studio_table)
    return jnp.concatenate([user_part, small_part], axis=1)


# speedup vs baseline: 1.3062x; 1.0030x over previous
import functools

import jax
import jax.numpy as jnp
from jax import lax
from jax.experimental import pallas as pl
from jax.experimental.pallas import tpu as pltpu
from jax.experimental.pallas import tpu_sc as plsc

EMBED = 64
BATCH = 16384

_info = plsc.get_sparse_core_info()
_NW = _info.num_cores * _info.num_subcores
_BPW = BATCH // _NW
_CH = 128

_mesh = plsc.VectorSubcoreMesh(core_axis_name="c", subcore_axis_name="s")


@functools.partial(
    pl.kernel,
    mesh=_mesh,
    out_type=jax.ShapeDtypeStruct((BATCH, EMBED), jnp.float32),
    scratch_types=[
        pltpu.VMEM((_BPW,), jnp.int32),
        [pltpu.VMEM((_CH, EMBED), jnp.float32) for _ in range(2)],
        [pltpu.SemaphoreType.DMA for _ in range(2)],
    ],
    compiler_params=pltpu.CompilerParams(use_tc_tiling_on_sc=True,
                                         needs_layout_passes=False),
    name="user_field",
)
def _user_field(uid, ut, out, idx_v, rows_v, sem):
    wid = lax.axis_index("s") * _info.num_cores + lax.axis_index("c")
    base = wid * _BPW
    pltpu.sync_copy(uid.at[pl.ds(base, _BPW)], idx_v)

    def issue(g, _, c, buf):
        v = idx_v[pl.ds(c * _CH + g * 16, 16)]
        for lane in range(16):
            pltpu.async_copy(
                ut.at[pl.ds(v[lane], 1), :],
                rows_v[buf].at[pl.ds(g * 16 + lane, 1), :],
                sem[buf])
        return 0

    def drain(g, _, buf):
        for lane in range(16):
            pltpu.make_async_copy(
                ut.at[pl.ds(0, 1), :],
                rows_v[0].at[pl.ds(0, 1), :],
                sem[buf]).wait()
        return 0

    def chunk(c):
        lax.fori_loop(0, _CH // 16, functools.partial(issue, c=c, buf=c % 2), 0)

    n_chunks = _BPW // _CH
    chunk(0)
    for c in range(n_chunks):
        if c + 1 < n_chunks:
            chunk(c + 1)
        lax.fori_loop(0, _CH // 16, functools.partial(drain, buf=c % 2), 0)
        pltpu.sync_copy(rows_v[c % 2], out.at[pl.ds(base + c * _CH, _CH), :])


@functools.partial(
    pl.kernel,
    mesh=_mesh,
    out_type=jax.ShapeDtypeStruct((BATCH, 4 * EMBED), jnp.float32),
    scratch_types=[
        [pltpu.VMEM((_BPW,), jnp.int32) for _ in range(4)],
        [pltpu.VMEM((_BPW, EMBED), jnp.float32) for _ in range(2)],
        pltpu.SemaphoreType.DMA,
    ],
    compiler_params=pltpu.CompilerParams(use_tc_tiling_on_sc=False),
    name="small_fields",
)
def _small_fields(ep, pop, yr, st, et, pt, yt, stt, out, idx_v, rows_v, gsem):
    wid = lax.axis_index("s") * _info.num_cores + lax.axis_index("c")
    base = wid * _BPW
    idx_hbm = [ep, pop, yr, st]
    tables = [et, pt, yt, stt]
    for t in range(4):
        pltpu.sync_copy(idx_hbm[t].at[pl.ds(base, _BPW)], idx_v[t])

    def start_gather(t, buf):
        return pltpu.async_copy(tables[t].at[idx_v[t]], rows_v[buf], gsem)

    cp = start_gather(0, 0)
    for t in range(4):
        cp.wait()
        if t + 1 < 4:
            nxt = start_gather(t + 1, (t + 1) % 2)
        pltpu.sync_copy(
            rows_v[t % 2],
            out.at[pl.ds(base, _BPW), pl.ds(t * EMBED, EMBED)])
        if t + 1 < 4:
            cp = nxt


def kernel(user_id, episodes, popularity, year, studio,
           user_table, episodes_table, popularity_table, year_table, studio_table):
    user_part = _user_field(user_id, user_table)
    small_part = _small_fields(episodes, popularity, year, studio,
                               episodes_table, popularity_table, year_table,
                               studio_table)
    return jnp.concatenate([user_part.T, small_part.T], axis=0).T
